# Initial kernel scaffold; baseline (speedup 1.0000x reference)
import jax
import jax.numpy as jnp
from jax.experimental import pallas as pl


def kernel(context_repr, cluster_centers):
    raise NotImplementedError("write your pallas kernel here")


# speedup vs baseline: 1.1588x; 1.1588x over previous
import functools

import jax
import jax.numpy as jnp
from jax.experimental import pallas as pl
from jax.experimental.pallas import tpu as pltpu
from jax.experimental.pallas import tpu_sc as plsc

N = 16384
K = 8192
D = 64
TM = 1024
KB = 2048
NB = N // TM
GW = 128

_EPS = 1e-12


def _argmin_kernel(x_ref, c_ref, a2_ref, b2_ref, idx_ref):
    xb = x_ref[...].astype(jnp.bfloat16)
    a2 = a2_ref[...]
    best = jnp.full((TM, 1), jnp.inf, jnp.float32)
    bidx = jnp.zeros((TM, 1), jnp.int32)
    for kc in range(K // KB):
        cb = c_ref[pl.ds(kc * KB, KB), :].astype(jnp.bfloat16)
        b2 = b2_ref[:, pl.ds(kc * KB, KB)]
        dots = jax.lax.dot_general(
            xb, cb, (((1,), (1,)), ((), ())),
            preferred_element_type=jnp.float32)
        d2 = (a2 + b2) - 2.0 * dots
        d2 = jnp.maximum(d2, 0.0)
        dist = jnp.where(d2 > 0.0, d2 * jax.lax.rsqrt(d2), 0.0)
        m = jnp.min(dist, axis=1, keepdims=True)
        lane = jax.lax.broadcasted_iota(jnp.int32, (TM, KB), 1) + kc * KB
        cand = jnp.where(dist == m, lane, jnp.int32(2**31 - 1))
        arg = jnp.min(cand, axis=1, keepdims=True)
        take = m < best
        bidx = jnp.where(take, arg, bidx)
        best = jnp.where(take, m, best)
    idx_ref[0, 0, :] = bidx[:, 0]


def _compute_indices(xn, cn, a2, b2):
    idx3 = pl.pallas_call(
        _argmin_kernel,
        grid=(NB,),
        in_specs=[
            pl.BlockSpec((TM, D), lambda i: (i, 0)),
            pl.BlockSpec((K, D), lambda i: (0, 0)),
            pl.BlockSpec((TM, 1), lambda i: (i, 0)),
            pl.BlockSpec((1, K), lambda i: (0, 0)),
        ],
        out_specs=pl.BlockSpec((1, 1, TM), lambda i: (i, 0, 0)),
        out_shape=jax.ShapeDtypeStruct((NB, 1, TM), jnp.int32),
    )(xn, cn, a2, b2)
    return idx3.reshape(N)


def _sc_gather(cluster_centers, indices):
    table = jnp.pad(cluster_centers, ((0, 0), (0, 128 - D)))
    ind2 = indices.reshape(1, N)
    mesh = plsc.VectorSubcoreMesh(core_axis_name="core",
                                  subcore_axis_name="subcore")

    @functools.partial(
        pl.kernel,
        out_type=jax.ShapeDtypeStruct((N, 128), cluster_centers.dtype),
        mesh=mesh)
    def gather_kernel(x_hbm, i_hbm, o_hbm):
        def body(i_vmem, o_vmem):
            pltpu.sync_copy(x_hbm.at[i_vmem.at[0]], o_vmem)

        pltpu.emit_pipeline(
            body,
            grid=(N // GW,),
            in_specs=[pl.BlockSpec((1, GW), index_map=lambda i: (0, i))],
            out_specs=[pl.BlockSpec((GW, 128), index_map=lambda i: (i, 0))],
            core_axis_name="subcore",
            dimension_semantics=(pltpu.PARALLEL,),
        )(i_hbm, o_hbm)

    return gather_kernel(table, ind2)[:, :D]


def kernel(context_repr, cluster_centers):
    xnorm = jnp.linalg.norm(context_repr, ord=2, axis=1, keepdims=True)
    xn = context_repr / jnp.maximum(xnorm, _EPS)
    cnorm = jnp.linalg.norm(cluster_centers, ord=2, axis=1, keepdims=True)
    cn = cluster_centers / jnp.maximum(cnorm, _EPS)
    a2 = jnp.sum(xn * xn, axis=1, keepdims=True)
    b2 = jnp.sum(cn * cn, axis=1)[None, :]
    indices = _compute_indices(xn, cn, a2, b2)
    return _sc_gather(cluster_centers, indices)
